# Initial kernel scaffold; baseline (speedup 1.0000x reference)
import jax
import jax.numpy as jnp
from jax.experimental import pallas as pl


def kernel(indices, values):
    raise NotImplementedError("write your pallas kernel here")


# speedup vs baseline: 4.9606x; 4.9606x over previous
import functools

import jax
import jax.numpy as jnp
from jax import lax
from jax.experimental import pallas as pl
from jax.experimental.pallas import tpu as pltpu
from jax.experimental.pallas import tpu_sc as plsc

N = 4096
NNZ = 167772
NC = 2
NS = 16
LANES = 16

CHUNK_ROWS = 256
NUM_CHUNKS = N // CHUNK_ROWS
PASSES = NUM_CHUNKS // NC
CHUNK_WORDS = CHUNK_ROWS * N
TRASH_PER_TILE = 16
ACC_WORDS = CHUNK_WORDS + NS * TRASH_PER_TILE

DMA_B = 128
E_CHUNKS = 82
E = E_CHUNKS * DMA_B
NNZ_P = NS * E

STAGE = 8192
WB_SLICE = CHUNK_WORDS // NS
WB_STEPS = WB_SLICE // STAGE


def _sc_body(rows_hbm, cols_hbm, vals_hbm, out_hbm,
             rows_v, cols_v, vals_v, idx_v, zbuf, wstage, acc):
    c = lax.axis_index("c")
    s = lax.axis_index("s")
    trash = jnp.int32(CHUNK_WORDS) + s * TRASH_PER_TILE

    pltpu.sync_copy(rows_hbm.at[s], rows_v)
    pltpu.sync_copy(cols_hbm.at[s], cols_v)
    pltpu.sync_copy(vals_hbm.at[s], vals_v)

    def zb_body(j, carry):
        zbuf[pl.ds(j * LANES, LANES)] = jnp.zeros((LANES,), jnp.float32)
        return carry

    lax.fori_loop(0, STAGE // LANES, zb_body, 0)

    for p in range(PASSES):
        chunk = p * NC + c
        row_base = chunk * CHUNK_ROWS

        w0 = s * WB_SLICE
        for i in range(WB_STEPS):
            pltpu.sync_copy(zbuf, acc.at[pl.ds(w0 + i * STAGE, STAGE)])
        pltpu.sync_copy(zbuf.at[pl.ds(0, TRASH_PER_TILE)],
                        acc.at[pl.ds(trash, TRASH_PER_TILE)])
        plsc.subcore_barrier()

        def idx_body(j, carry):
            for k in range(DMA_B // LANES):
                e = j * DMA_B + k * LANES
                r = rows_v[pl.ds(e, LANES)]
                col = cols_v[pl.ds(e, LANES)]
                local = (r - row_base) * N + col
                ok = (r >= row_base) & (r < row_base + CHUNK_ROWS)
                idx_v[j, pl.ds(k * LANES, LANES)] = jnp.where(ok, local, trash)
            return carry

        lax.fori_loop(0, E_CHUNKS, idx_body, 0)

        def scat_body(j, carry):
            pltpu.sync_copy(vals_v.at[j], acc.at[idx_v.at[j]], add=True)
            return carry

        lax.fori_loop(0, E_CHUNKS, scat_body, 0)
        plsc.subcore_barrier()

        for i in range(WB_STEPS):
            pltpu.sync_copy(acc.at[pl.ds(w0 + i * STAGE, STAGE)], wstage)
            pltpu.sync_copy(
                wstage,
                out_hbm.at[pl.ds(chunk * CHUNK_WORDS + w0 + i * STAGE, STAGE)])
        plsc.subcore_barrier()


@jax.jit
def _dispatch(rows_p, cols_p, vals_p):
    mesh = plsc.VectorSubcoreMesh(core_axis_name="c", subcore_axis_name="s",
                                  num_cores=NC, num_subcores=NS)
    return pl.kernel(
        _sc_body,
        out_type=jax.ShapeDtypeStruct((N * N,), jnp.float32),
        mesh=mesh,
        scratch_types=[
            pltpu.VMEM((E,), jnp.int32),
            pltpu.VMEM((E,), jnp.int32),
            pltpu.VMEM((E_CHUNKS, DMA_B), jnp.float32),
            pltpu.VMEM((E_CHUNKS, DMA_B), jnp.int32),
            pltpu.VMEM((STAGE,), jnp.float32),
            pltpu.VMEM((STAGE,), jnp.float32),
            pltpu.VMEM_SHARED((ACC_WORDS,), jnp.float32),
        ],
    )(rows_p, cols_p, vals_p)


def kernel(indices, values):
    idx32 = indices.astype(jnp.int32)
    rows = idx32[:, 0]
    cols = idx32[:, 1]
    pad = NNZ_P - NNZ
    rows_p = jnp.concatenate(
        [rows, jnp.full((pad,), 2 * N, jnp.int32)]).reshape(NS, E)
    cols_p = jnp.concatenate(
        [cols, jnp.zeros((pad,), jnp.int32)]).reshape(NS, E)
    vals_p = jnp.concatenate(
        [values.astype(jnp.float32), jnp.zeros((pad,), jnp.float32)]
    ).reshape(NS, E_CHUNKS, DMA_B)
    out = _dispatch(rows_p, cols_p, vals_p)
    return out.reshape(N, N)


# speedup vs baseline: 5.4646x; 1.1016x over previous
import functools

import jax
import jax.numpy as jnp
from jax import lax
from jax.experimental import pallas as pl
from jax.experimental.pallas import tpu as pltpu
from jax.experimental.pallas import tpu_sc as plsc

N = 4096
NNZ = 167772
NC = 2
NS = 16
LANES = 16

CHUNK_ROWS = 256
NUM_CHUNKS = N // CHUNK_ROWS
PASSES = NUM_CHUNKS // NC
CHUNK_WORDS = CHUNK_ROWS * N
TRASH_PER_TILE = 16
ACC_WORDS = CHUNK_WORDS + NS * TRASH_PER_TILE

DMA_B = 128
E_CHUNKS = 82
E = E_CHUNKS * DMA_B
NNZ_P = NS * E

STAGE = 8192
WB_SLICE = CHUNK_WORDS // NS
WB_STEPS = WB_SLICE // STAGE
IDX_SUB = E_CHUNKS // WB_STEPS


def _sc_body(off_hbm, vals_hbm, out_hbm,
             off_v, vals_v, idx_v, zbuf, ws0, ws1,
             acc, sem_s, sem_rd, sem_wr, sem_z):
    c = lax.axis_index("c")
    s = lax.axis_index("s")
    trash = jnp.int32(CHUNK_WORDS) + s * TRASH_PER_TILE
    ws = [ws0, ws1]

    pltpu.sync_copy(off_hbm.at[s], off_v)
    pltpu.sync_copy(vals_hbm.at[s], vals_v)

    def zb_body(j, carry):
        zbuf[pl.ds(j * LANES, LANES)] = jnp.zeros((LANES,), jnp.float32)
        return carry

    lax.fori_loop(0, STAGE // LANES, zb_body, 0)

    def make_idx_body(p):
        lo = (p * NC + c) * CHUNK_WORDS

        def idx_body(j, carry):
            for k in range(DMA_B // LANES):
                e = j * DMA_B + k * LANES
                d = off_v[pl.ds(e, LANES)] - lo
                ok = d.astype(jnp.uint32) < jnp.uint32(CHUNK_WORDS)
                idx_v[j, pl.ds(k * LANES, LANES)] = jnp.where(ok, d, trash)
            return carry

        return idx_body

    w0 = s * WB_SLICE
    zh = []
    for i in range(WB_STEPS):
        zh.append(pltpu.async_copy(
            zbuf, acc.at[pl.ds(w0 + i * STAGE, STAGE)], sem_z))
    lax.fori_loop(0, E_CHUNKS, make_idx_body(0), 0)
    for h in zh:
        h.wait()
    plsc.subcore_barrier()

    SCAT_W = 8
    for p in range(PASSES):
        hs = []
        for j in range(E_CHUNKS):
            hs.append(pltpu.async_copy(vals_v.at[j], acc.at[idx_v.at[j]],
                                       sem_s, add=True))
            if j >= SCAT_W:
                hs[j - SCAT_W].wait()
        for h in hs[E_CHUNKS - SCAT_W:]:
            h.wait()
        plsc.subcore_barrier()

        chunk = p * NC + c
        out0 = chunk * CHUNK_WORDS + w0
        rd = [None] * WB_STEPS
        wr = [None] * WB_STEPS
        zr = [None] * WB_STEPS
        rd[0] = pltpu.async_copy(acc.at[pl.ds(w0, STAGE)], ws[0], sem_rd)
        for i in range(WB_STEPS):
            rd[i].wait()
            if i >= 1:
                wr[i - 1].wait()
            wr[i] = pltpu.async_copy(
                ws[i % 2], out_hbm.at[pl.ds(out0 + i * STAGE, STAGE)], sem_wr)
            if p + 1 < PASSES:
                if i >= 2:
                    zr[i - 2].wait()
                zr[i] = pltpu.async_copy(
                    zbuf, acc.at[pl.ds(w0 + i * STAGE, STAGE)], sem_z)
            if i + 1 < WB_STEPS:
                rd[i + 1] = pltpu.async_copy(
                    acc.at[pl.ds(w0 + (i + 1) * STAGE, STAGE)],
                    ws[(i + 1) % 2], sem_rd)
            if p + 1 < PASSES:
                j0 = i * IDX_SUB
                j1 = (i + 1) * IDX_SUB if i + 1 < WB_STEPS else E_CHUNKS
                lax.fori_loop(j0, j1, make_idx_body(p + 1), 0)
        wr[WB_STEPS - 1].wait()
        if p + 1 < PASSES:
            zr[WB_STEPS - 2].wait()
            zr[WB_STEPS - 1].wait()
            plsc.subcore_barrier()


@jax.jit
def _dispatch(off_p, vals_p):
    mesh = plsc.VectorSubcoreMesh(core_axis_name="c", subcore_axis_name="s",
                                  num_cores=NC, num_subcores=NS)
    return pl.kernel(
        _sc_body,
        out_type=jax.ShapeDtypeStruct((N * N,), jnp.float32),
        mesh=mesh,
        scratch_types=[
            pltpu.VMEM((E,), jnp.int32),
            pltpu.VMEM((E_CHUNKS, DMA_B), jnp.float32),
            pltpu.VMEM((E_CHUNKS, DMA_B), jnp.int32),
            pltpu.VMEM((STAGE,), jnp.float32),
            pltpu.VMEM((STAGE,), jnp.float32),
            pltpu.VMEM((STAGE,), jnp.float32),
            pltpu.VMEM_SHARED((ACC_WORDS,), jnp.float32),
            pltpu.SemaphoreType.DMA,
            pltpu.SemaphoreType.DMA,
            pltpu.SemaphoreType.DMA,
            pltpu.SemaphoreType.DMA,
        ],
    )(off_p, vals_p)


def kernel(indices, values):
    idx32 = indices.astype(jnp.int32)
    off = idx32[:, 0] * N + idx32[:, 1]
    pad = NNZ_P - NNZ
    off_p = jnp.concatenate(
        [off, jnp.full((pad,), 2 * N * N, jnp.int32)]).reshape(NS, E)
    vals_p = jnp.concatenate(
        [values.astype(jnp.float32), jnp.zeros((pad,), jnp.float32)]
    ).reshape(NS, E_CHUNKS, DMA_B)
    out = _dispatch(off_p, vals_p)
    return out.reshape(N, N)


# speedup vs baseline: 7.4763x; 1.3681x over previous
import functools

import jax
import jax.numpy as jnp
from jax import lax
from jax.experimental import pallas as pl
from jax.experimental.pallas import tpu as pltpu
from jax.experimental.pallas import tpu_sc as plsc

N = 4096
NNZ = 167772
NC = 2
NS = 16
LANES = 16

CHUNK_ROWS = 256
NUM_CHUNKS = N // CHUNK_ROWS
PASSES = NUM_CHUNKS // NC
CHUNK_WORDS = CHUNK_ROWS * N
TRASH_WORDS = 4096
ACC_WORDS = CHUNK_WORDS + TRASH_WORDS

DMA_B = 128
E_CHUNKS = 82
E = E_CHUNKS * DMA_B
NNZ_P = NS * E

STAGE = 8192
ZSTAGE = 4096
WB_SLICE = CHUNK_WORDS // NS
WB_STEPS = WB_SLICE // STAGE
IDX_SUB = E_CHUNKS // WB_STEPS


def _sc_body(off_hbm, vals_hbm, out_hbm,
             off_v, vals_v, idx_v, zbuf, ws0, ws1,
             acc, sem_s, sem_rd, sem_wr, sem_z):
    c = lax.axis_index("c")
    s = lax.axis_index("s")
    ws = [ws0, ws1]

    eh0 = pltpu.async_copy(off_hbm.at[s], off_v, sem_rd)
    eh1 = pltpu.async_copy(vals_hbm.at[s], vals_v, sem_wr)

    def zb_body(j, carry):
        zbuf[pl.ds(j * LANES, LANES)] = jnp.zeros((LANES,), jnp.float32)
        return carry

    lax.fori_loop(0, ZSTAGE // LANES, zb_body, 0)
    eh0.wait()
    eh1.wait()

    def make_idx_body(p):
        lo = (p * NC + c) * CHUNK_WORDS

        def idx_body(j, carry):
            for k in range(DMA_B // LANES):
                e = j * DMA_B + k * LANES
                off = off_v[pl.ds(e, LANES)]
                d = off - lo
                ok = d.astype(jnp.uint32) < jnp.uint32(CHUNK_WORDS)
                t = jnp.int32(CHUNK_WORDS) + (off & (TRASH_WORDS - 1))
                idx_v[j, pl.ds(k * LANES, LANES)] = jnp.where(ok, d, t)
            return carry

        return idx_body

    w0 = s * WB_SLICE
    zh = []
    for i in range(2 * WB_STEPS):
        zh.append(pltpu.async_copy(
            zbuf, acc.at[pl.ds(w0 + i * ZSTAGE, ZSTAGE)], sem_z))
        if i >= 4:
            zh[i - 4].wait()
    lax.fori_loop(0, E_CHUNKS, make_idx_body(0), 0)
    for h in zh[2 * WB_STEPS - 4:]:
        h.wait()
    plsc.subcore_barrier()

    SCAT_W = 8
    for p in range(PASSES):
        hs = []
        for j in range(E_CHUNKS):
            hs.append(pltpu.async_copy(vals_v.at[j], acc.at[idx_v.at[j]],
                                       sem_s, add=True))
            if j >= SCAT_W:
                hs[j - SCAT_W].wait()
        for h in hs[E_CHUNKS - SCAT_W:]:
            h.wait()
        plsc.subcore_barrier()

        chunk = p * NC + c
        out0 = chunk * CHUNK_WORDS + w0
        last = p + 1 == PASSES
        rd = [None] * WB_STEPS
        wr = [None] * WB_STEPS
        zr = [None] * (2 * WB_STEPS)
        rd[0] = pltpu.async_copy(acc.at[pl.ds(w0, STAGE)], ws[0], sem_rd)
        for i in range(WB_STEPS):
            rd[i].wait()
            if i >= 1:
                wr[i - 1].wait()
            wr[i] = pltpu.async_copy(
                ws[i % 2], out_hbm.at[pl.ds(out0 + i * STAGE, STAGE)], sem_wr)
            if not last:
                if i >= 2:
                    zr[2 * i - 4].wait()
                    zr[2 * i - 3].wait()
                zr[2 * i] = pltpu.async_copy(
                    zbuf, acc.at[pl.ds(w0 + 2 * i * ZSTAGE, ZSTAGE)], sem_z)
                zr[2 * i + 1] = pltpu.async_copy(
                    zbuf, acc.at[pl.ds(w0 + (2 * i + 1) * ZSTAGE, ZSTAGE)],
                    sem_z)
            if i + 1 < WB_STEPS:
                rd[i + 1] = pltpu.async_copy(
                    acc.at[pl.ds(w0 + (i + 1) * STAGE, STAGE)],
                    ws[(i + 1) % 2], sem_rd)
            if not last:
                j0 = i * IDX_SUB
                j1 = (i + 1) * IDX_SUB if i + 1 < WB_STEPS else E_CHUNKS
                lax.fori_loop(j0, j1, make_idx_body(p + 1), 0)
        wr[WB_STEPS - 1].wait()
        if not last:
            zr[2 * WB_STEPS - 4].wait()
            zr[2 * WB_STEPS - 3].wait()
            zr[2 * WB_STEPS - 2].wait()
            zr[2 * WB_STEPS - 1].wait()
            plsc.subcore_barrier()


@jax.jit
def _dispatch(off_p, vals_p):
    mesh = plsc.VectorSubcoreMesh(core_axis_name="c", subcore_axis_name="s",
                                  num_cores=NC, num_subcores=NS)
    return pl.kernel(
        _sc_body,
        out_type=jax.ShapeDtypeStruct((N * N,), jnp.float32),
        mesh=mesh,
        scratch_types=[
            pltpu.VMEM((E,), jnp.int32),
            pltpu.VMEM((E_CHUNKS, DMA_B), jnp.float32),
            pltpu.VMEM((E_CHUNKS, DMA_B), jnp.int32),
            pltpu.VMEM((ZSTAGE,), jnp.float32),
            pltpu.VMEM((STAGE,), jnp.float32),
            pltpu.VMEM((STAGE,), jnp.float32),
            pltpu.VMEM_SHARED((ACC_WORDS,), jnp.float32),
            pltpu.SemaphoreType.DMA,
            pltpu.SemaphoreType.DMA,
            pltpu.SemaphoreType.DMA,
            pltpu.SemaphoreType.DMA,
        ],
    )(off_p, vals_p)


def kernel(indices, values):
    idx32 = indices.astype(jnp.int32)
    off = idx32[:, 0] * N + idx32[:, 1]
    pad = NNZ_P - NNZ
    off_p = jnp.concatenate(
        [off, jnp.full((pad,), 2 * N * N, jnp.int32)]).reshape(NS, E)
    vals_p = jnp.concatenate(
        [values.astype(jnp.float32), jnp.zeros((pad,), jnp.float32)]
    ).reshape(NS, E_CHUNKS, DMA_B)
    out = _dispatch(off_p, vals_p)
    return out.reshape(N, N)


# speedup vs baseline: 7.4963x; 1.0027x over previous
import functools

import jax
import jax.numpy as jnp
from jax import lax
from jax.experimental import pallas as pl
from jax.experimental.pallas import tpu as pltpu
from jax.experimental.pallas import tpu_sc as plsc

N = 4096
NNZ = 167772
NC = 2
NS = 16
LANES = 16

CHUNK_ROWS = 256
NUM_CHUNKS = N // CHUNK_ROWS
PASSES = NUM_CHUNKS // NC
CHUNK_WORDS = CHUNK_ROWS * N
TRASH_WORDS = 16384
ACC_WORDS = CHUNK_WORDS + TRASH_WORDS

DMA_B = 128
E_CHUNKS = 82
E = E_CHUNKS * DMA_B
NNZ_P = NS * E

STAGE = 8192
ZSTAGE = 4096
WB_SLICE = CHUNK_WORDS // NS
WB_STEPS = WB_SLICE // STAGE
IDX_SUB = E_CHUNKS // WB_STEPS


def _sc_body(off_hbm, vals_hbm, out_hbm,
             off_v, vals_v, idx_v, zbuf, ws0, ws1,
             acc, sem_s, sem_rd, sem_wr, sem_z):
    c = lax.axis_index("c")
    s = lax.axis_index("s")
    ws = [ws0, ws1]

    eh0 = pltpu.async_copy(off_hbm.at[s], off_v, sem_rd)
    eh1 = pltpu.async_copy(vals_hbm.at[s], vals_v, sem_wr)

    def zb_body(j, carry):
        zbuf[pl.ds(j * LANES, LANES)] = jnp.zeros((LANES,), jnp.float32)
        return carry

    lax.fori_loop(0, ZSTAGE // LANES, zb_body, 0)
    eh0.wait()
    eh1.wait()

    def make_idx_body(p):
        lo = (p * NC + c) * CHUNK_WORDS

        def idx_body(j, carry):
            for k in range(DMA_B // LANES):
                e = j * DMA_B + k * LANES
                off = off_v[pl.ds(e, LANES)]
                d = off - lo
                ok = d.astype(jnp.uint32) < jnp.uint32(CHUNK_WORDS)
                t = jnp.int32(CHUNK_WORDS) + (off & (TRASH_WORDS - 1))
                idx_v[j, pl.ds(k * LANES, LANES)] = jnp.where(ok, d, t)
            return carry

        return idx_body

    w0 = s * WB_SLICE
    zh = []
    for i in range(2 * WB_STEPS):
        zh.append(pltpu.async_copy(
            zbuf, acc.at[pl.ds(w0 + i * ZSTAGE, ZSTAGE)], sem_z))
        if i >= 4:
            zh[i - 4].wait()
    lax.fori_loop(0, E_CHUNKS, make_idx_body(0), 0)
    for h in zh[2 * WB_STEPS - 4:]:
        h.wait()
    plsc.subcore_barrier()

    SCAT_W = 16
    for p in range(PASSES):
        hs = []
        for j in range(E_CHUNKS):
            hs.append(pltpu.async_copy(vals_v.at[j], acc.at[idx_v.at[j]],
                                       sem_s, add=True))
            if j >= SCAT_W:
                hs[j - SCAT_W].wait()
        for h in hs[E_CHUNKS - SCAT_W:]:
            h.wait()
        plsc.subcore_barrier()

        chunk = p * NC + c
        out0 = chunk * CHUNK_WORDS + w0
        last = p + 1 == PASSES
        rd = [None] * WB_STEPS
        wr = [None] * WB_STEPS
        zr = [None] * (2 * WB_STEPS)
        rd[0] = pltpu.async_copy(acc.at[pl.ds(w0, STAGE)], ws[0], sem_rd)
        for i in range(WB_STEPS):
            rd[i].wait()
            if i >= 1:
                wr[i - 1].wait()
            wr[i] = pltpu.async_copy(
                ws[i % 2], out_hbm.at[pl.ds(out0 + i * STAGE, STAGE)], sem_wr)
            if not last:
                if i >= 2:
                    zr[2 * i - 4].wait()
                    zr[2 * i - 3].wait()
                zr[2 * i] = pltpu.async_copy(
                    zbuf, acc.at[pl.ds(w0 + 2 * i * ZSTAGE, ZSTAGE)], sem_z)
                zr[2 * i + 1] = pltpu.async_copy(
                    zbuf, acc.at[pl.ds(w0 + (2 * i + 1) * ZSTAGE, ZSTAGE)],
                    sem_z)
            if i + 1 < WB_STEPS:
                rd[i + 1] = pltpu.async_copy(
                    acc.at[pl.ds(w0 + (i + 1) * STAGE, STAGE)],
                    ws[(i + 1) % 2], sem_rd)
            if not last:
                j0 = i * IDX_SUB
                j1 = (i + 1) * IDX_SUB if i + 1 < WB_STEPS else E_CHUNKS
                lax.fori_loop(j0, j1, make_idx_body(p + 1), 0)
        wr[WB_STEPS - 1].wait()
        if not last:
            zr[2 * WB_STEPS - 4].wait()
            zr[2 * WB_STEPS - 3].wait()
            zr[2 * WB_STEPS - 2].wait()
            zr[2 * WB_STEPS - 1].wait()
            plsc.subcore_barrier()


@jax.jit
def _dispatch(off_p, vals_p):
    mesh = plsc.VectorSubcoreMesh(core_axis_name="c", subcore_axis_name="s",
                                  num_cores=NC, num_subcores=NS)
    return pl.kernel(
        _sc_body,
        out_type=jax.ShapeDtypeStruct((N * N,), jnp.float32),
        mesh=mesh,
        scratch_types=[
            pltpu.VMEM((E,), jnp.int32),
            pltpu.VMEM((E_CHUNKS, DMA_B), jnp.float32),
            pltpu.VMEM((E_CHUNKS, DMA_B), jnp.int32),
            pltpu.VMEM((ZSTAGE,), jnp.float32),
            pltpu.VMEM((STAGE,), jnp.float32),
            pltpu.VMEM((STAGE,), jnp.float32),
            pltpu.VMEM_SHARED((ACC_WORDS,), jnp.float32),
            pltpu.SemaphoreType.DMA,
            pltpu.SemaphoreType.DMA,
            pltpu.SemaphoreType.DMA,
            pltpu.SemaphoreType.DMA,
        ],
    )(off_p, vals_p)


def kernel(indices, values):
    idx32 = indices.astype(jnp.int32)
    off = idx32[:, 0] * N + idx32[:, 1]
    pad = NNZ_P - NNZ
    off_p = jnp.concatenate(
        [off, jnp.full((pad,), 2 * N * N, jnp.int32)]).reshape(NS, E)
    vals_p = jnp.concatenate(
        [values.astype(jnp.float32), jnp.zeros((pad,), jnp.float32)]
    ).reshape(NS, E_CHUNKS, DMA_B)
    out = _dispatch(off_p, vals_p)
    return out.reshape(N, N)
